# baseline (speedup 1.0000x reference)
import functools
import jax
import jax.numpy as jnp
from jax import lax
from jax.experimental import pallas as pl
from jax.experimental.pallas import tpu as pltpu
from jax.experimental.pallas import tpu_sc as plsc

_EMBED_DIM = 128
_HIDDEN_DIM = 1024
_NUM_CHORDS = 1000
_BATCH = 4096
_HIST = 20
_SPANS = (2, 4, 6, 8)
_BLOCK_B = 4096


def _make_sc_gather(total_rows, dim, chunk):
    info = plsc.get_sparse_core_info()
    nw = info.num_cores * info.num_subcores
    rows_per_w = total_rows // nw
    n_chunks = rows_per_w // chunk
    mesh = plsc.VectorSubcoreMesh(core_axis_name="c", subcore_axis_name="s")

    @functools.partial(
        pl.kernel,
        mesh=mesh,
        out_type=jax.ShapeDtypeStruct((total_rows, dim), jnp.float32),
        scratch_types=[
            pltpu.VMEM((chunk,), jnp.int32),
            pltpu.VMEM((chunk, dim), jnp.float32),
            pltpu.SemaphoreType.DMA,
        ],
    )
    def gather_kernel(table_hbm, idx_hbm, out_hbm, idx_v, rows_v, sem):
        wid = lax.axis_index("s") * info.num_cores + lax.axis_index("c")
        base = wid * rows_per_w

        def body(ci, _):
            off = base + ci * chunk
            pltpu.sync_copy(idx_hbm.at[pl.ds(off, chunk)], idx_v)
            pltpu.async_copy(table_hbm.at[idx_v], rows_v, sem).wait()
            pltpu.sync_copy(rows_v, out_hbm.at[pl.ds(off, chunk)])
            return ()

        lax.fori_loop(0, n_chunks, body, (), unroll=False)

    return gather_kernel


def _fuse_weights_kernel(w1_ref, w2t_ref, b1_ref, b2_ref, wct_ref, bcc_ref):
    wct_ref[...] = lax.dot_general(
        w2t_ref[...], w1_ref[...], (((1,), (1,)), ((), ())),
        preferred_element_type=jnp.float32).astype(jnp.bfloat16)
    bcc_ref[...] = jnp.sum(w2t_ref[...] * b1_ref[...], axis=1,
                           keepdims=True) + b2_ref[...]


def _fuse_weights(W1, W2, b1, b2):
    return pl.pallas_call(
        _fuse_weights_kernel,
        out_shape=(
            jax.ShapeDtypeStruct((_NUM_CHORDS, _EMBED_DIM), jnp.bfloat16),
            jax.ShapeDtypeStruct((_NUM_CHORDS, 1), jnp.float32),
        ),
    )(W1, jnp.transpose(W2), b1.reshape(1, _HIDDEN_DIM),
      b2.reshape(_NUM_CHORDS, 1))


def _matmul_t_kernel(x_ref, wct_ref, bcc_ref, o_ref):
    o_ref[0] = lax.dot_general(
        wct_ref[...], x_ref[0].astype(jnp.bfloat16), (((1,), (1,)), ((), ())),
        preferred_element_type=jnp.float32) + bcc_ref[...]


def _matmul_t_first(X3, WcT, bcc, l_span):
    grid = (l_span, _BATCH // _BLOCK_B)
    return pl.pallas_call(
        _matmul_t_kernel,
        grid=grid,
        in_specs=[
            pl.BlockSpec((1, _BLOCK_B, _EMBED_DIM), lambda l, j: (l, j, 0)),
            pl.BlockSpec((_NUM_CHORDS, _EMBED_DIM), lambda l, j: (0, 0)),
            pl.BlockSpec((_NUM_CHORDS, 1), lambda l, j: (0, 0)),
        ],
        out_specs=pl.BlockSpec((1, _NUM_CHORDS, _BLOCK_B),
                               lambda l, j: (l, 0, j)),
        out_shape=jax.ShapeDtypeStruct((_HIST, _NUM_CHORDS, _BATCH),
                                       jnp.float32),
        compiler_params=pltpu.CompilerParams(
            dimension_semantics=("parallel", "parallel"),
        ),
    )(X3, WcT, bcc)


def _matmul_t_next(out_buf, X3, WcT, bcc, l_off, l_span):
    grid = (l_span, _BATCH // _BLOCK_B)

    def _noop_out_kernel(o_buf_ref, x_ref, wct_ref, bcc_ref, o_ref):
        del o_buf_ref
        _matmul_t_kernel(x_ref, wct_ref, bcc_ref, o_ref)

    return pl.pallas_call(
        _noop_out_kernel,
        grid=grid,
        in_specs=[
            pl.BlockSpec(memory_space=pl.ANY),
            pl.BlockSpec((1, _BLOCK_B, _EMBED_DIM), lambda l, j: (l, j, 0)),
            pl.BlockSpec((_NUM_CHORDS, _EMBED_DIM), lambda l, j: (0, 0)),
            pl.BlockSpec((_NUM_CHORDS, 1), lambda l, j: (0, 0)),
        ],
        out_specs=pl.BlockSpec((1, _NUM_CHORDS, _BLOCK_B),
                               lambda l, j: (l + l_off, 0, j)),
        out_shape=jax.ShapeDtypeStruct((_HIST, _NUM_CHORDS, _BATCH),
                                       jnp.float32),
        input_output_aliases={0: 0},
        compiler_params=pltpu.CompilerParams(
            dimension_semantics=("parallel", "parallel"),
        ),
    )(out_buf, X3, WcT, bcc)


def _inner_chunk(rows_per_w):
    n_inner = -(-rows_per_w // 256)
    while rows_per_w % n_inner:
        n_inner += 1
    return rows_per_w // n_inner


@jax.jit
def kernel(input_notes, emb, W1, b1, W2, b2):
    batch, hist = input_notes.shape
    idx = jnp.transpose(input_notes).reshape(-1).astype(jnp.int32)
    WcT, bcc = _fuse_weights(W1, W2, b1, b2)
    xs = []
    row_off = 0
    for span in _SPANS:
        rows = span * batch
        gather = _make_sc_gather(rows, _EMBED_DIM,
                                 chunk=_inner_chunk(rows // 32))
        xs.append(
            gather(emb, lax.dynamic_slice_in_dim(idx, row_off, rows))
            .reshape(span, batch, _EMBED_DIM))
        row_off += rows
    out_t = _matmul_t_first(xs[0], WcT, bcc, _SPANS[0])
    l_off = _SPANS[0]
    for k in range(1, len(_SPANS)):
        out_t = _matmul_t_next(out_t, xs[k], WcT, bcc, l_off, _SPANS[k])
        l_off += _SPANS[k]
    return jnp.transpose(out_t, (2, 0, 1))


# speedup vs baseline: 1.0041x; 1.0041x over previous
import functools
import jax
import jax.numpy as jnp
from jax import lax
from jax.experimental import pallas as pl
from jax.experimental.pallas import tpu as pltpu
from jax.experimental.pallas import tpu_sc as plsc

_EMBED_DIM = 128
_HIDDEN_DIM = 1024
_NUM_CHORDS = 1000
_BATCH = 4096
_HIST = 20
_SPANS = (2, 4, 6, 8)
_BLOCK_B = 4096


def _make_sc_gather(total_rows, dim, chunk):
    info = plsc.get_sparse_core_info()
    nw = info.num_cores * info.num_subcores
    rows_per_w = total_rows // nw
    n_chunks = rows_per_w // chunk
    mesh = plsc.VectorSubcoreMesh(core_axis_name="c", subcore_axis_name="s")

    @functools.partial(
        pl.kernel,
        mesh=mesh,
        out_type=jax.ShapeDtypeStruct((total_rows, dim), jnp.float32),
        scratch_types=[
            pltpu.VMEM((chunk,), jnp.int32),
            pltpu.VMEM((chunk, dim), jnp.float32),
            pltpu.SemaphoreType.DMA,
        ],
    )
    def gather_kernel(table_hbm, idx_hbm, out_hbm, idx_v, rows_v, sem):
        wid = lax.axis_index("s") * info.num_cores + lax.axis_index("c")
        base = wid * rows_per_w

        def body(ci, _):
            off = base + ci * chunk
            pltpu.sync_copy(idx_hbm.at[pl.ds(off, chunk)], idx_v)
            pltpu.async_copy(table_hbm.at[idx_v], rows_v, sem).wait()
            pltpu.sync_copy(rows_v, out_hbm.at[pl.ds(off, chunk)])
            return ()

        lax.fori_loop(0, n_chunks, body, (), unroll=False)

    return gather_kernel


def _fuse_weights_kernel(w1_ref, w2t_ref, b1_ref, b2_ref, wct_ref, bcc_ref):
    wct_ref[...] = lax.dot_general(
        w2t_ref[...], w1_ref[...], (((1,), (1,)), ((), ())),
        preferred_element_type=jnp.float32).astype(jnp.bfloat16)
    bcc_ref[...] = jnp.sum(w2t_ref[...] * b1_ref[...], axis=1,
                           keepdims=True) + b2_ref[...]


def _fuse_weights(W1, W2, b1, b2):
    return pl.pallas_call(
        _fuse_weights_kernel,
        out_shape=(
            jax.ShapeDtypeStruct((_NUM_CHORDS, _EMBED_DIM), jnp.bfloat16),
            jax.ShapeDtypeStruct((_NUM_CHORDS, 1), jnp.float32),
        ),
    )(W1, jnp.transpose(W2), b1.reshape(1, _HIDDEN_DIM),
      b2.reshape(_NUM_CHORDS, 1))


def _matmul_t_kernel(x_ref, wct_ref, bcc_ref, o_ref):
    o_ref[0] = lax.dot_general(
        wct_ref[...], x_ref[0].astype(jnp.bfloat16), (((1,), (1,)), ((), ())),
        preferred_element_type=jnp.float32) + bcc_ref[...]


def _matmul_t_first(X3, WcT, bcc, l_span):
    grid = (l_span, _BATCH // _BLOCK_B)
    return pl.pallas_call(
        _matmul_t_kernel,
        grid=grid,
        in_specs=[
            pl.BlockSpec((1, _BLOCK_B, _EMBED_DIM), lambda l, j: (l, j, 0)),
            pl.BlockSpec((_NUM_CHORDS, _EMBED_DIM), lambda l, j: (0, 0)),
            pl.BlockSpec((_NUM_CHORDS, 1), lambda l, j: (0, 0)),
        ],
        out_specs=pl.BlockSpec((1, _NUM_CHORDS, _BLOCK_B),
                               lambda l, j: (l, 0, j)),
        out_shape=jax.ShapeDtypeStruct((_HIST, _NUM_CHORDS, _BATCH),
                                       jnp.float32),
        compiler_params=pltpu.CompilerParams(
            dimension_semantics=("arbitrary", "arbitrary"),
        ),
    )(X3, WcT, bcc)


def _matmul_t_next(out_buf, X3, WcT, bcc, l_off, l_span):
    grid = (l_span, _BATCH // _BLOCK_B)

    def _noop_out_kernel(o_buf_ref, x_ref, wct_ref, bcc_ref, o_ref):
        del o_buf_ref
        _matmul_t_kernel(x_ref, wct_ref, bcc_ref, o_ref)

    return pl.pallas_call(
        _noop_out_kernel,
        grid=grid,
        in_specs=[
            pl.BlockSpec(memory_space=pl.ANY),
            pl.BlockSpec((1, _BLOCK_B, _EMBED_DIM), lambda l, j: (l, j, 0)),
            pl.BlockSpec((_NUM_CHORDS, _EMBED_DIM), lambda l, j: (0, 0)),
            pl.BlockSpec((_NUM_CHORDS, 1), lambda l, j: (0, 0)),
        ],
        out_specs=pl.BlockSpec((1, _NUM_CHORDS, _BLOCK_B),
                               lambda l, j: (l + l_off, 0, j)),
        out_shape=jax.ShapeDtypeStruct((_HIST, _NUM_CHORDS, _BATCH),
                                       jnp.float32),
        input_output_aliases={0: 0},
        compiler_params=pltpu.CompilerParams(
            dimension_semantics=("arbitrary", "arbitrary"),
        ),
    )(out_buf, X3, WcT, bcc)


def _inner_chunk(rows_per_w):
    n_inner = -(-rows_per_w // 256)
    while rows_per_w % n_inner:
        n_inner += 1
    return rows_per_w // n_inner


@jax.jit
def kernel(input_notes, emb, W1, b1, W2, b2):
    batch, hist = input_notes.shape
    idx = jnp.transpose(input_notes).reshape(-1).astype(jnp.int32)
    WcT, bcc = _fuse_weights(W1, W2, b1, b2)
    xs = []
    row_off = 0
    for span in _SPANS:
        rows = span * batch
        gather = _make_sc_gather(rows, _EMBED_DIM,
                                 chunk=_inner_chunk(rows // 32))
        xs.append(
            gather(emb, lax.dynamic_slice_in_dim(idx, row_off, rows))
            .reshape(span, batch, _EMBED_DIM))
        row_off += rows
    out_t = _matmul_t_first(xs[0], WcT, bcc, _SPANS[0])
    l_off = _SPANS[0]
    for k in range(1, len(_SPANS)):
        out_t = _matmul_t_next(out_t, xs[k], WcT, bcc, l_off, _SPANS[k])
        l_off += _SPANS[k]
    return jnp.transpose(out_t, (2, 0, 1))
